# baseline (speedup 1.0000x reference)
import functools

import jax
import jax.numpy as jnp
from jax import lax
from jax.experimental import pallas as pl
from jax.experimental.pallas import tpu as pltpu
from jax.experimental.pallas import tpu_sc as plsc

NUM_CORES = 2
NUM_SUBCORES = 16
NW = NUM_CORES * NUM_SUBCORES
LANE = 128
BL = 128
PACK = 4


def _build_gather(batch: int, seq: int, d_model: int):
    mesh = plsc.VectorSubcoreMesh(
        core_axis_name="c", subcore_axis_name="s",
        num_cores=NUM_CORES, num_subcores=NUM_SUBCORES)

    @functools.partial(
        pl.kernel,
        out_type=jax.ShapeDtypeStruct((seq, d_model, batch), jnp.float32),
        mesh=mesh,
        scratch_types=[
            pltpu.VMEM((seq, BL), jnp.int32),
            pltpu.VMEM((seq, BL), jnp.int32),
            pltpu.VMEM((4, BL, LANE), jnp.float32),
            pltpu.VMEM((4, d_model, BL), jnp.float32),
            pltpu.SemaphoreType.DMA,
            pltpu.SemaphoreType.DMA,
            pltpu.SemaphoreType.DMA,
            pltpu.SemaphoreType.DMA,
            pltpu.SemaphoreType.DMA,
            pltpu.SemaphoreType.DMA,
            pltpu.SemaphoreType.DMA,
            pltpu.SemaphoreType.DMA,
        ],
        compiler_params=pltpu.CompilerParams(needs_layout_passes=False),
    )
    def gather_kernel(xt_hbm, tp_hbm, out_hbm, xv, xq, rbuf, tbuf, *sems):
        wid = lax.axis_index("s") * NUM_CORES + lax.axis_index("c")
        b0 = wid * BL
        pltpu.sync_copy(xt_hbm.at[:, pl.ds(b0, BL)], xv)
        gsems = sems[:4]
        ssems = sems[4:]
        iota = lax.iota(jnp.int32, 16)
        nring = 4
        nstep = seq // nring
        ntail = seq - nstep * nring

        @pl.loop(0, seq)
        def _(s):
            for lb in range(BL // 16):
                v = xv[s, pl.ds(16 * lb, 16)]
                xq[s, pl.ds(16 * lb, 16)] = lax.shift_right_logical(v, 2)
                xv[s, pl.ds(16 * lb, 16)] = lax.shift_left(
                    jnp.bitwise_and(v, 3), 5)

        def gather_desc(s, b):
            return pltpu.make_async_copy(
                tp_hbm.at[xq.at[s]], rbuf.at[b], gsems[b])

        def store_desc(s, b):
            return pltpu.make_async_copy(
                tbuf.at[b], out_hbm.at[s].at[:, pl.ds(b0, BL)], ssems[b])

        def transpose(s, b):
            src = rbuf.at[b]
            dst = tbuf.at[b]
            for lb in range(BL // 16):
                rows = iota + (16 * lb)
                qcol = xv[s, pl.ds(16 * lb, 16)]
                for d in range(d_model):
                    dst[d, pl.ds(16 * lb, 16)] = plsc.load_gather(
                        src, [rows, qcol + d])

        for b in range(nring):
            gather_desc(b, b).start()

        @pl.loop(0, nstep)
        def _(h):
            h0 = h * nring
            for b in range(nring):
                s = h0 + b
                gather_desc(s, b).wait()

                @pl.when(h > 0)
                def _():
                    store_desc(s, b).wait()
                transpose(s, b)
                store_desc(s, b).start()

                @pl.when(s + nring < seq)
                def _():
                    gather_desc(s + nring, b).start()

        for b in range(nring):
            s_prev = nstep * nring - nring + b
            if b < ntail:
                st = nstep * nring + b
                store_desc(st, b).wait()
                gather_desc(st, b).wait()
                transpose(st, b)
                store_desc(st, b).start()
                store_desc(st, b).wait()
            else:
                store_desc(s_prev, b).wait()

    return gather_kernel


def kernel(x, embedding_matrix):
    batch, seq = x.shape
    vocab, d_model = embedding_matrix.shape
    tp = embedding_matrix.reshape(vocab * d_model // LANE, LANE)
    gather = _build_gather(batch, seq, d_model)
    out_t = gather(x.T, tp)
    return out_t.transpose(2, 0, 1)
